# Initial kernel scaffold; baseline (speedup 1.0000x reference)
import jax
import jax.numpy as jnp
from jax.experimental import pallas as pl


def kernel(node_type_idx, edge_index, query_edges, type_embed, Wg1, bg1, Wg2, bg2, We1, be1, We2, be2):
    raise NotImplementedError("write your pallas kernel here")


# speedup vs baseline: 5.0258x; 5.0258x over previous
import functools

import jax
import jax.numpy as jnp
from jax import lax
from jax.experimental import pallas as pl
from jax.experimental.pallas import tpu as pltpu
from jax.experimental.pallas import tpu_sc as plsc

NC = 2
NS = 16
NW = NC * NS
L = 16


def _make_hist(N, E, T):
    CH = 128
    assert E % CH == 0
    n_chunks = E // CH
    iters = (n_chunks + NW - 1) // NW
    NK = N * T
    assert NK % NS == 0
    per_tile = NK // NS
    mesh = plsc.VectorSubcoreMesh(core_axis_name="c", subcore_axis_name="s", num_cores=NC, num_subcores=NS)

    @functools.partial(
        pl.kernel,
        out_type=jax.ShapeDtypeStruct((NC, NK), jnp.float32),
        mesh=mesh,
        compiler_params=pltpu.CompilerParams(use_tc_tiling_on_sc=False, needs_layout_passes=False),
        scratch_types=[
            pltpu.VMEM((N,), jnp.int32),
            pltpu.VMEM((1, CH), jnp.int32),
            pltpu.VMEM((1, CH), jnp.int32),
            pltpu.VMEM((CH,), jnp.float32),
            pltpu.VMEM_SHARED((NK,), jnp.float32),
        ],
    )
    def hist(types_hbm, src_hbm, dst_hbm, zeros_hbm, out_hbm,
             types_v, srcb, dstb, onesb, hist_sh):
        cid = lax.axis_index("c")
        sid = lax.axis_index("s")
        wid = sid * NC + cid

        pltpu.sync_copy(types_hbm, types_v)
        for k in range(CH // L):
            onesb[pl.ds(k * L, L)] = jnp.full((L,), 1.0, jnp.float32)
        pltpu.sync_copy(zeros_hbm.at[pl.ds(sid * per_tile, per_tile)],
                        hist_sh.at[pl.ds(sid * per_tile, per_tile)])
        plsc.subcore_barrier()

        def body(j, _):
            c = wid + NW * j

            @pl.when(c < n_chunks)
            def _():
                base = c * CH
                pltpu.sync_copy(src_hbm.at[pl.ds(base, CH)], srcb.at[0])
                pltpu.sync_copy(dst_hbm.at[pl.ds(base, CH)], dstb.at[0])
                for k in range(CH // L):
                    sl = pl.ds(k * L, L)
                    t16 = plsc.load_gather(types_v, [srcb[0, sl]])
                    dstb[0, sl] = dstb[0, sl] * T + t16
                pltpu.sync_copy(onesb, hist_sh.at[dstb.at[0]], add=True)

            return ()

        lax.fori_loop(0, iters, body, ())
        plsc.subcore_barrier()
        pltpu.sync_copy(hist_sh.at[pl.ds(sid * per_tile, per_tile)],
                        out_hbm.at[cid, pl.ds(sid * per_tile, per_tile)])

    return hist


def _make_agg2(N, E, HH):
    CH = 128
    assert E % CH == 0
    n_chunks = E // CH
    iters = (n_chunks + NS - 1) // NS
    assert N % NS == 0
    rows_per_tile = N // NS
    nfull = rows_per_tile // CH
    rem = rows_per_tile - nfull * CH
    mesh = plsc.VectorSubcoreMesh(core_axis_name="c", subcore_axis_name="s", num_cores=NC, num_subcores=NS)

    @functools.partial(
        pl.kernel,
        out_type=jax.ShapeDtypeStruct((NC, N, HH), jnp.float32),
        mesh=mesh,
        compiler_params=pltpu.CompilerParams(use_tc_tiling_on_sc=False, needs_layout_passes=False),
        scratch_types=[
            pltpu.VMEM((1, CH), jnp.int32),
            pltpu.VMEM((1, CH), jnp.int32),
            pltpu.VMEM((CH, HH), jnp.float32),
            pltpu.VMEM_SHARED((N, HH), jnp.float32),
        ],
    )
    def agg2(x1s_hbm, src_hbm, dst_hbm, out_hbm, srcb, dstb, rows, acc_sh):
        cid = lax.axis_index("c")
        sid = lax.axis_index("s")
        row0 = sid * rows_per_tile
        half0 = cid * N

        for r in range(CH):
            for k in range(HH // L):
                rows[r, pl.ds(k * L, L)] = jnp.zeros((L,), jnp.float32)
        for q in range(nfull):
            pltpu.sync_copy(rows, acc_sh.at[pl.ds(row0 + q * CH, CH)])
        if rem:
            pltpu.sync_copy(rows.at[pl.ds(0, rem)],
                            acc_sh.at[pl.ds(row0 + nfull * CH, rem)])
        plsc.subcore_barrier()

        def body(j, _):
            c = sid + NS * j

            @pl.when(c < n_chunks)
            def _():
                base = c * CH
                pltpu.sync_copy(src_hbm.at[pl.ds(base, CH)], srcb.at[0])
                pltpu.sync_copy(dst_hbm.at[pl.ds(base, CH)], dstb.at[0])
                for k in range(CH // L):
                    sl = pl.ds(k * L, L)
                    srcb[0, sl] = srcb[0, sl] + half0
                pltpu.sync_copy(x1s_hbm.at[srcb.at[0]], rows)
                pltpu.sync_copy(rows, acc_sh.at[dstb.at[0]], add=True)

            return ()

        lax.fori_loop(0, iters, body, ())
        plsc.subcore_barrier()
        pltpu.sync_copy(acc_sh.at[pl.ds(row0, rows_per_tile)],
                        out_hbm.at[cid, pl.ds(row0, rows_per_tile)])

    return agg2


def _make_qgather(N, EQ, H):
    CH = 80
    assert EQ % CH == 0
    n_chunks = EQ // CH
    iters = (n_chunks + NW - 1) // NW
    mesh = plsc.VectorSubcoreMesh(core_axis_name="c", subcore_axis_name="s", num_cores=NC, num_subcores=NS)

    @functools.partial(
        pl.kernel,
        out_type=jax.ShapeDtypeStruct((EQ, H), jnp.float32),
        mesh=mesh,
        compiler_params=pltpu.CompilerParams(use_tc_tiling_on_sc=False, needs_layout_passes=False),
        scratch_types=[
            pltpu.VMEM((CH,), jnp.int32),
            pltpu.VMEM((CH,), jnp.int32),
            pltpu.VMEM((CH, H), jnp.float32),
            pltpu.VMEM((CH, H), jnp.float32),
        ],
    )
    def qgather(qa_hbm, qb_hbm, q0_hbm, q1_hbm, s_hbm, q0b, q1b, ua, ub):
        cid = lax.axis_index("c")
        sid = lax.axis_index("s")
        wid = sid * NC + cid

        def body(j, _):
            c = wid + NW * j

            @pl.when(c < n_chunks)
            def _():
                base = c * CH
                pltpu.sync_copy(q0_hbm.at[pl.ds(base, CH)], q0b)
                pltpu.sync_copy(q1_hbm.at[pl.ds(base, CH)], q1b)
                pltpu.sync_copy(qa_hbm.at[q0b], ua)
                pltpu.sync_copy(qb_hbm.at[q1b], ub)
                for r in range(CH):
                    for k in range(H // L):
                        sl = pl.ds(k * L, L)
                        ua[r, sl] = ua[r, sl] + ub[r, sl]
                pltpu.sync_copy(ua, s_hbm.at[pl.ds(base, CH)])

            return ()

        lax.fori_loop(0, iters, body, ())

    return qgather


def _mlp1_body(h0_ref, h1_ref, nt_ref, te_ref, wg1_ref, bg1_ref,
               x1a_ref, x1b_ref, den_ref):
    counts = h0_ref[...] + h1_ref[...]
    deg = jnp.sum(counts, axis=1, keepdims=True)
    denom = jnp.maximum(deg, 1.0)
    onehot = (nt_ref[...] == lax.broadcasted_iota(jnp.int32, (1, 8), 1)
              ).astype(jnp.float32)
    cprime = onehot + counts / denom
    te = te_ref[...]
    xn = cprime[:, 0:1] * te[0:1, :]
    for t in range(1, 8):
        xn = xn + cprime[:, t:t + 1] * te[t:t + 1, :]
    x1 = jnp.maximum(
        jnp.dot(xn, wg1_ref[...], preferred_element_type=jnp.float32)
        + bg1_ref[...], 0.0)
    x1a_ref[...] = x1[:, :32]
    x1b_ref[...] = x1[:, 32:]
    den_ref[...] = denom


def _mlp2_body(x1a_ref, x1b_ref, aga_ref, agb_ref, den_ref, wg2_ref, bg2_ref,
               wa_ref, wb_ref, be1_ref, qa_ref, qb_ref):
    x1 = jnp.concatenate([x1a_ref[...], x1b_ref[...]], axis=1)
    agg = jnp.concatenate([aga_ref[...], agb_ref[...]], axis=1)
    xn = x1 + agg / den_ref[...]
    x2 = jnp.maximum(
        jnp.dot(xn, wg2_ref[...], preferred_element_type=jnp.float32)
        + bg2_ref[...], 0.0)
    qa_ref[...] = jnp.dot(x2, wa_ref[...], preferred_element_type=jnp.float32)
    qb_ref[...] = (jnp.dot(x2, wb_ref[...], preferred_element_type=jnp.float32)
                   + be1_ref[...])


def _score_body(s_ref, w2_ref, be2_ref, o_ref):
    h = jnp.maximum(s_ref[...], 0.0)
    o_ref[...] = (jnp.dot(h, w2_ref[...], preferred_element_type=jnp.float32)
                  + be2_ref[...])


def _full(shape):
    return pl.BlockSpec(shape, lambda i: (0,) * len(shape))


def kernel(node_type_idx, edge_index, query_edges, type_embed,
           Wg1, bg1, Wg2, bg2, We1, be1, We2, be2):
    N = node_type_idx.shape[0]
    E = edge_index.shape[1]
    EQ = query_edges.shape[1]
    H = type_embed.shape[1]
    T = type_embed.shape[0]
    HH = H // 2
    R = 2000
    assert N % R == 0 and EQ % R == 0

    src = edge_index[0]
    dst = edge_index[1]
    q0 = query_edges[0]
    q1 = query_edges[1]

    hist = _make_hist(N, E, T)(
        node_type_idx, src, dst, jnp.zeros((N * T,), jnp.float32))
    h0 = hist[0].reshape(N, T)
    h1 = hist[1].reshape(N, T)

    nt2 = node_type_idx.reshape(N, 1)
    x1a, x1b, denom = pl.pallas_call(
        _mlp1_body,
        grid=(N // R,),
        in_specs=[
            pl.BlockSpec((R, T), lambda i: (i, 0)),
            pl.BlockSpec((R, T), lambda i: (i, 0)),
            pl.BlockSpec((R, 1), lambda i: (i, 0)),
            _full((T, H)),
            _full((H, H)),
            _full((1, H)),
        ],
        out_specs=[
            pl.BlockSpec((R, HH), lambda i: (i, 0)),
            pl.BlockSpec((R, HH), lambda i: (i, 0)),
            pl.BlockSpec((R, 1), lambda i: (i, 0)),
        ],
        out_shape=[
            jax.ShapeDtypeStruct((N, HH), jnp.float32),
            jax.ShapeDtypeStruct((N, HH), jnp.float32),
            jax.ShapeDtypeStruct((N, 1), jnp.float32),
        ],
    )(h0, h1, nt2, type_embed, Wg1, bg1.reshape(1, H))

    x1s = jnp.concatenate([x1a, x1b], axis=0)
    agg = _make_agg2(N, E, HH)(x1s, src, dst)
    agga, aggb = agg[0], agg[1]

    qa, qb = pl.pallas_call(
        _mlp2_body,
        grid=(N // R,),
        in_specs=[
            pl.BlockSpec((R, HH), lambda i: (i, 0)),
            pl.BlockSpec((R, HH), lambda i: (i, 0)),
            pl.BlockSpec((R, HH), lambda i: (i, 0)),
            pl.BlockSpec((R, HH), lambda i: (i, 0)),
            pl.BlockSpec((R, 1), lambda i: (i, 0)),
            _full((H, H)),
            _full((1, H)),
            _full((H, H)),
            _full((H, H)),
            _full((1, H)),
        ],
        out_specs=[
            pl.BlockSpec((R, H), lambda i: (i, 0)),
            pl.BlockSpec((R, H), lambda i: (i, 0)),
        ],
        out_shape=[
            jax.ShapeDtypeStruct((N, H), jnp.float32),
            jax.ShapeDtypeStruct((N, H), jnp.float32),
        ],
    )(x1a, x1b, agga, aggb, denom, Wg2, bg2.reshape(1, H),
      We1[:H], We1[H:], be1.reshape(1, H))

    s = _make_qgather(N, EQ, H)(qa, qb, q0, q1)

    logits = pl.pallas_call(
        _score_body,
        grid=(EQ // R,),
        in_specs=[
            pl.BlockSpec((R, H), lambda i: (i, 0)),
            _full((H, 1)),
            _full((1, 1)),
        ],
        out_specs=pl.BlockSpec((R, 1), lambda i: (i, 0)),
        out_shape=jax.ShapeDtypeStruct((EQ, 1), jnp.float32),
    )(s, We2, be2.reshape(1, 1))

    return logits.reshape(EQ)


# speedup vs baseline: 7.9933x; 1.5905x over previous
import functools

import jax
import jax.numpy as jnp
from jax import lax
from jax.experimental import pallas as pl
from jax.experimental.pallas import tpu as pltpu
from jax.experimental.pallas import tpu_sc as plsc

NC = 2
NS = 16
NW = NC * NS
L = 16
CH = 128

_SC_PARAMS = pltpu.CompilerParams(use_tc_tiling_on_sc=False,
                                  needs_layout_passes=False)


def _mesh():
    return plsc.VectorSubcoreMesh(core_axis_name="c", subcore_axis_name="s",
                                  num_cores=NC, num_subcores=NS)


def _make_hist(N, T, n_chunks):
    assert n_chunks % NW == 0
    nc_w = n_chunks // NW
    assert nc_w % 4 == 0
    NK = N * T
    assert NK % NS == 0
    per_tile = NK // NS

    @functools.partial(
        pl.kernel,
        out_type=jax.ShapeDtypeStruct((NC, NK), jnp.float32),
        mesh=_mesh(),
        compiler_params=_SC_PARAMS,
        scratch_types=[
            pltpu.VMEM((N,), jnp.int32),
            pltpu.VMEM((CH,), jnp.float32),
            pltpu.VMEM_SHARED((NK + T,), jnp.float32),
        ] + [pltpu.VMEM((2, CH), jnp.int32) for _ in range(4)]
          + [pltpu.SemaphoreType.DMA for _ in range(8)],
    )
    def hist(types_hbm, earr_hbm, zeros_hbm, out_hbm,
             types_v, onesb, hist_sh,
             eb0, eb1, eb2, eb3, si0, si1, si2, si3, ss0, ss1, ss2, ss3):
        ebuf = [eb0, eb1, eb2, eb3]
        sem_i = [si0, si1, si2, si3]
        sem_s = [ss0, ss1, ss2, ss3]
        cid = lax.axis_index("c")
        sid = lax.axis_index("s")
        wid = sid * NC + cid
        c0 = wid * nc_w

        pltpu.sync_copy(types_hbm, types_v)
        for k in range(CH // L):
            onesb[pl.ds(k * L, L)] = jnp.full((L,), 1.0, jnp.float32)
        pltpu.sync_copy(zeros_hbm.at[pl.ds(sid * per_tile, per_tile)],
                        hist_sh.at[pl.ds(sid * per_tile, per_tile)])
        plsc.subcore_barrier()

        pltpu.async_copy(earr_hbm.at[c0], ebuf[0], sem_i[0])
        pltpu.async_copy(earr_hbm.at[c0 + 1], ebuf[1], sem_i[1])

        def body(jj, _):
            for p in range(4):
                j = jj * 4 + p
                c = c0 + j
                pf = (p + 2) % 4
                pltpu.make_async_copy(earr_hbm.at[c], ebuf[p], sem_i[p]).wait()
                for k in range(CH // L):
                    sl = pl.ds(k * L, L)
                    t16 = plsc.load_gather(types_v, [ebuf[p][0, sl]])
                    ebuf[p][1, sl] = ebuf[p][1, sl] * T + t16

                @pl.when(j >= 2)
                def _():
                    pltpu.make_async_copy(
                        onesb, hist_sh.at[ebuf[pf].at[1]], sem_s[pf]).wait()

                @pl.when(j + 2 < nc_w)
                def _():
                    pltpu.async_copy(earr_hbm.at[c + 2], ebuf[pf], sem_i[pf])

                pltpu.async_copy(onesb, hist_sh.at[ebuf[p].at[1]],
                                 sem_s[p], add=True)
            return ()

        lax.fori_loop(0, nc_w // 4, body, ())
        for p in ((nc_w - 2) % 4, (nc_w - 1) % 4):
            pltpu.make_async_copy(onesb, hist_sh.at[ebuf[p].at[1]],
                                  sem_s[p]).wait()
        plsc.subcore_barrier()
        pltpu.sync_copy(hist_sh.at[pl.ds(sid * per_tile, per_tile)],
                        out_hbm.at[cid, pl.ds(sid * per_tile, per_tile)])

    return hist


def _make_agg2(N, HH, n_chunks):
    assert n_chunks % NS == 0
    nc_t = n_chunks // NS
    assert nc_t % 4 == 0
    assert N % NS == 0
    rows_per_tile = N // NS
    nfull = rows_per_tile // CH
    rem = rows_per_tile - nfull * CH

    @functools.partial(
        pl.kernel,
        out_type=jax.ShapeDtypeStruct((NC, N, HH), jnp.float32),
        mesh=_mesh(),
        compiler_params=_SC_PARAMS,
        scratch_types=[
            pltpu.VMEM_SHARED((N + 8, HH), jnp.float32),
        ] + [pltpu.VMEM((2, CH), jnp.int32) for _ in range(4)]
          + [pltpu.VMEM((CH, HH), jnp.float32) for _ in range(4)]
          + [pltpu.SemaphoreType.DMA for _ in range(9)],
    )
    def agg2(x1s_hbm, earr_hbm, out_hbm, acc_sh,
             eb0, eb1, eb2, eb3, rw0, rw1, rw2, rw3,
             si0, si1, si2, si3, sg, ss0, ss1, ss2, ss3):
        ebuf = [eb0, eb1, eb2, eb3]
        rows = [rw0, rw1, rw2, rw3]
        sem_i = [si0, si1, si2, si3]
        sem_s = [ss0, ss1, ss2, ss3]
        cid = lax.axis_index("c")
        sid = lax.axis_index("s")
        row0 = sid * rows_per_tile
        half0 = cid * N
        c0 = sid * nc_t

        for r in range(CH):
            for k in range(HH // L):
                rw0[r, pl.ds(k * L, L)] = jnp.zeros((L,), jnp.float32)
        for q in range(nfull):
            pltpu.sync_copy(rw0, acc_sh.at[pl.ds(row0 + q * CH, CH)])
        if rem:
            pltpu.sync_copy(rw0.at[pl.ds(0, rem)],
                            acc_sh.at[pl.ds(row0 + nfull * CH, rem)])
        plsc.subcore_barrier()

        pltpu.async_copy(earr_hbm.at[c0], ebuf[0], sem_i[0])
        pltpu.async_copy(earr_hbm.at[c0 + 1], ebuf[1], sem_i[1])

        def body(jj, _):
            for p in range(4):
                j = jj * 4 + p
                c = c0 + j
                pf = (p + 2) % 4
                pltpu.make_async_copy(earr_hbm.at[c], ebuf[p], sem_i[p]).wait()
                for k in range(CH // L):
                    sl = pl.ds(k * L, L)
                    ebuf[p][0, sl] = ebuf[p][0, sl] + half0
                g = pltpu.async_copy(x1s_hbm.at[ebuf[p].at[0]], rows[p], sg)

                @pl.when(j >= 2)
                def _():
                    pltpu.make_async_copy(
                        rows[pf], acc_sh.at[ebuf[pf].at[1]], sem_s[pf]).wait()

                @pl.when(j + 2 < nc_t)
                def _():
                    pltpu.async_copy(earr_hbm.at[c + 2], ebuf[pf], sem_i[pf])

                g.wait()
                pltpu.async_copy(rows[p], acc_sh.at[ebuf[p].at[1]],
                                 sem_s[p], add=True)
            return ()

        lax.fori_loop(0, nc_t // 4, body, ())
        for p in ((nc_t - 2) % 4, (nc_t - 1) % 4):
            pltpu.make_async_copy(rows[p], acc_sh.at[ebuf[p].at[1]],
                                  sem_s[p]).wait()
        plsc.subcore_barrier()
        pltpu.sync_copy(acc_sh.at[pl.ds(row0, rows_per_tile)],
                        out_hbm.at[cid, pl.ds(row0, rows_per_tile)])

    return agg2


def _make_qgather(EQ_pad, H, n_chunks, CQ):
    assert n_chunks % NW == 0
    nc_w = n_chunks // NW
    assert nc_w % 4 == 0

    @functools.partial(
        pl.kernel,
        out_type=jax.ShapeDtypeStruct((EQ_pad, H), jnp.float32),
        mesh=_mesh(),
        compiler_params=_SC_PARAMS,
        scratch_types=[pltpu.VMEM((2, CQ), jnp.int32) for _ in range(4)]
                      + [pltpu.VMEM((CQ, H), jnp.float32) for _ in range(8)]
                      + [pltpu.SemaphoreType.DMA for _ in range(9)],
    )
    def qgather(qa_hbm, qb_hbm, qarr_hbm, s_hbm,
                qb0, qb1, qb2, qb3, ua0, ua1, ua2, ua3, ub0, ub1, ub2, ub3,
                si0, si1, si2, si3, sg, sw0, sw1, sw2, sw3):
        qbuf = [qb0, qb1, qb2, qb3]
        ua = [ua0, ua1, ua2, ua3]
        ub = [ub0, ub1, ub2, ub3]
        sem_i = [si0, si1, si2, si3]
        sem_w = [sw0, sw1, sw2, sw3]
        cid = lax.axis_index("c")
        sid = lax.axis_index("s")
        wid = sid * NC + cid
        c0 = wid * nc_w

        pltpu.async_copy(qarr_hbm.at[c0], qbuf[0], sem_i[0])
        pltpu.async_copy(qarr_hbm.at[c0 + 1], qbuf[1], sem_i[1])

        def body(jj, _):
            for p in range(4):
                j = jj * 4 + p
                c = c0 + j
                pf = (p + 2) % 4
                pltpu.make_async_copy(qarr_hbm.at[c], qbuf[p], sem_i[p]).wait()
                ga = pltpu.async_copy(qa_hbm.at[qbuf[p].at[0]], ua[p], sg)
                gb = pltpu.async_copy(qb_hbm.at[qbuf[p].at[1]], ub[p], sg)

                @pl.when(j >= 2)
                def _():
                    pltpu.make_async_copy(
                        ua[pf], s_hbm.at[pl.ds((c - 2) * CQ, CQ)],
                        sem_w[pf]).wait()

                @pl.when(j + 2 < nc_w)
                def _():
                    pltpu.async_copy(qarr_hbm.at[c + 2], qbuf[pf], sem_i[pf])

                ga.wait()
                gb.wait()
                for r in range(CQ):
                    for k in range(H // L):
                        sl = pl.ds(k * L, L)
                        ua[p][r, sl] = ua[p][r, sl] + ub[p][r, sl]
                pltpu.async_copy(ua[p], s_hbm.at[pl.ds(c * CQ, CQ)], sem_w[p])
            return ()

        lax.fori_loop(0, nc_w // 4, body, ())
        for p in ((nc_w - 2) % 4, (nc_w - 1) % 4):
            c = c0 + nc_w - 1
            pltpu.make_async_copy(ua[p], s_hbm.at[pl.ds(c * CQ, CQ)],
                                  sem_w[p]).wait()

    return qgather


def _mlp1_body(h0_ref, h1_ref, nt_ref, te_ref, wg1_ref, bg1_ref,
               x1_ref, den_ref):
    counts = h0_ref[...] + h1_ref[...]
    deg = jnp.sum(counts, axis=1, keepdims=True)
    denom = jnp.maximum(deg, 1.0)
    onehot = (nt_ref[...] == lax.broadcasted_iota(jnp.int32, (1, 8), 1)
              ).astype(jnp.float32)
    cprime = onehot + counts / denom
    te = te_ref[...]
    xn = cprime[:, 0:1] * te[0:1, :]
    for t in range(1, 8):
        xn = xn + cprime[:, t:t + 1] * te[t:t + 1, :]
    x1 = jnp.maximum(
        jnp.dot(xn, wg1_ref[...], preferred_element_type=jnp.float32)
        + bg1_ref[...], 0.0)
    hh = x1.shape[1] // 2
    x1_ref[...] = jnp.stack([x1[:, :hh], x1[:, hh:]], axis=0)
    den_ref[...] = denom


def _mlp2_body(x1_ref, agg_ref, den_ref, wg2_ref, bg2_ref,
               wa_ref, wb_ref, be1_ref, qa_ref, qb_ref):
    x1blk = x1_ref[...]
    aggblk = agg_ref[...]
    x1 = jnp.concatenate([x1blk[0], x1blk[1]], axis=1)
    agg = jnp.concatenate([aggblk[0], aggblk[1]], axis=1)
    xn = x1 + agg / den_ref[...]
    x2 = jnp.maximum(
        jnp.dot(xn, wg2_ref[...], preferred_element_type=jnp.float32)
        + bg2_ref[...], 0.0)
    qa_ref[...] = jnp.dot(x2, wa_ref[...], preferred_element_type=jnp.float32)
    qb_ref[...] = (jnp.dot(x2, wb_ref[...], preferred_element_type=jnp.float32)
                   + be1_ref[...])


def _score_body(s_ref, w2_ref, be2_ref, o_ref):
    h = jnp.maximum(s_ref[...], 0.0)
    o_ref[...] = (jnp.dot(h, w2_ref[...], preferred_element_type=jnp.float32)
                  + be2_ref[...])


def _full(shape):
    return pl.BlockSpec(shape, lambda i: (0,) * len(shape))


def kernel(node_type_idx, edge_index, query_edges, type_embed,
           Wg1, bg1, Wg2, bg2, We1, be1, We2, be2):
    N = node_type_idx.shape[0]
    E = edge_index.shape[1]
    EQ = query_edges.shape[1]
    H = type_embed.shape[1]
    T = type_embed.shape[0]
    HH = H // 2
    R = 2000
    assert N % R == 0

    n_chunks = -(-E // (CH * NS * 4)) * (NS * 4)
    assert n_chunks % NW == 0 and (n_chunks // NW) % 4 == 0
    E_pad = n_chunks * CH
    src_p = jnp.concatenate(
        [edge_index[0], jnp.zeros((E_pad - E,), jnp.int32)])
    dst_p = jnp.concatenate(
        [edge_index[1], jnp.full((E_pad - E,), N, jnp.int32)])
    earr = jnp.stack([src_p.reshape(n_chunks, CH),
                      dst_p.reshape(n_chunks, CH)], axis=1)

    CQ = 80
    nq_chunks = -(-EQ // (CQ * NW * 4)) * (NW * 4)
    EQ_pad = nq_chunks * CQ
    q0_p = jnp.concatenate(
        [query_edges[0], jnp.zeros((EQ_pad - EQ,), jnp.int32)])
    q1_p = jnp.concatenate(
        [query_edges[1], jnp.zeros((EQ_pad - EQ,), jnp.int32)])
    qarr = jnp.stack([q0_p.reshape(nq_chunks, CQ),
                      q1_p.reshape(nq_chunks, CQ)], axis=1)

    hist = _make_hist(N, T, n_chunks)(
        node_type_idx, earr, jnp.zeros((N * T,), jnp.float32))
    h0 = hist[0].reshape(N, T)
    h1 = hist[1].reshape(N, T)

    nt2 = node_type_idx.reshape(N, 1)
    x1, denom = pl.pallas_call(
        _mlp1_body,
        grid=(N // R,),
        in_specs=[
            pl.BlockSpec((R, T), lambda i: (i, 0)),
            pl.BlockSpec((R, T), lambda i: (i, 0)),
            pl.BlockSpec((R, 1), lambda i: (i, 0)),
            _full((T, H)),
            _full((H, H)),
            _full((1, H)),
        ],
        out_specs=[
            pl.BlockSpec((2, R, HH), lambda i: (0, i, 0)),
            pl.BlockSpec((R, 1), lambda i: (i, 0)),
        ],
        out_shape=[
            jax.ShapeDtypeStruct((2, N, HH), jnp.float32),
            jax.ShapeDtypeStruct((N, 1), jnp.float32),
        ],
    )(h0, h1, nt2, type_embed, Wg1, bg1.reshape(1, H))

    agg = _make_agg2(N, HH, n_chunks)(x1.reshape(2 * N, HH), earr)

    qa, qb = pl.pallas_call(
        _mlp2_body,
        grid=(N // R,),
        in_specs=[
            pl.BlockSpec((2, R, HH), lambda i: (0, i, 0)),
            pl.BlockSpec((2, R, HH), lambda i: (0, i, 0)),
            pl.BlockSpec((R, 1), lambda i: (i, 0)),
            _full((H, H)),
            _full((1, H)),
            _full((H, H)),
            _full((H, H)),
            _full((1, H)),
        ],
        out_specs=[
            pl.BlockSpec((R, H), lambda i: (i, 0)),
            pl.BlockSpec((R, H), lambda i: (i, 0)),
        ],
        out_shape=[
            jax.ShapeDtypeStruct((N, H), jnp.float32),
            jax.ShapeDtypeStruct((N, H), jnp.float32),
        ],
    )(x1, agg, denom, Wg2, bg2.reshape(1, H),
      We1[:H], We1[H:], be1.reshape(1, H))

    s = _make_qgather(EQ_pad, H, nq_chunks, CQ)(qa, qb, qarr)

    RQ = 2048
    assert EQ_pad % RQ == 0
    logits = pl.pallas_call(
        _score_body,
        grid=(EQ_pad // RQ,),
        in_specs=[
            pl.BlockSpec((RQ, H), lambda i: (i, 0)),
            _full((H, 1)),
            _full((1, 1)),
        ],
        out_specs=pl.BlockSpec((RQ, 1), lambda i: (i, 0)),
        out_shape=jax.ShapeDtypeStruct((EQ_pad, 1), jnp.float32),
    )(s, We2, be2.reshape(1, 1))

    return logits.reshape(EQ_pad)[:EQ]


# speedup vs baseline: 9.5439x; 1.1940x over previous
import functools

import jax
import jax.numpy as jnp
from jax import lax
from jax.experimental import pallas as pl
from jax.experimental.pallas import tpu as pltpu
from jax.experimental.pallas import tpu_sc as plsc

NC = 2
NS = 16
NW = NC * NS
L = 16
CH = 128

_SC_PARAMS = pltpu.CompilerParams(use_tc_tiling_on_sc=False,
                                  needs_layout_passes=False)


def _mesh():
    return plsc.VectorSubcoreMesh(core_axis_name="c", subcore_axis_name="s",
                                  num_cores=NC, num_subcores=NS)


def _make_hist(N, T, n_chunks):
    assert n_chunks % NW == 0
    nc_w = n_chunks // NW
    assert nc_w % 4 == 0
    NK = N * T
    assert NK % NS == 0
    per_tile = NK // NS

    @functools.partial(
        pl.kernel,
        out_type=jax.ShapeDtypeStruct((NC, NK), jnp.float32),
        mesh=_mesh(),
        compiler_params=_SC_PARAMS,
        scratch_types=[
            pltpu.VMEM((N,), jnp.int32),
            pltpu.VMEM((CH,), jnp.float32),
            pltpu.VMEM_SHARED((NK + T,), jnp.float32),
        ] + [pltpu.VMEM((2, CH), jnp.int32) for _ in range(4)]
          + [pltpu.SemaphoreType.DMA for _ in range(8)],
    )
    def hist(types_hbm, earr_hbm, zeros_hbm, out_hbm,
             types_v, onesb, hist_sh,
             eb0, eb1, eb2, eb3, si0, si1, si2, si3, ss0, ss1, ss2, ss3):
        ebuf = [eb0, eb1, eb2, eb3]
        sem_i = [si0, si1, si2, si3]
        sem_s = [ss0, ss1, ss2, ss3]
        cid = lax.axis_index("c")
        sid = lax.axis_index("s")
        wid = sid * NC + cid
        c0 = wid * nc_w

        pltpu.sync_copy(types_hbm, types_v)
        for k in range(CH // L):
            onesb[pl.ds(k * L, L)] = jnp.full((L,), 1.0, jnp.float32)
        pltpu.sync_copy(zeros_hbm.at[pl.ds(sid * per_tile, per_tile)],
                        hist_sh.at[pl.ds(sid * per_tile, per_tile)])
        plsc.subcore_barrier()

        pltpu.async_copy(earr_hbm.at[c0], ebuf[0], sem_i[0])
        pltpu.async_copy(earr_hbm.at[c0 + 1], ebuf[1], sem_i[1])

        def body(jj, _):
            for p in range(4):
                j = jj * 4 + p
                c = c0 + j
                pf = (p + 2) % 4
                pltpu.make_async_copy(earr_hbm.at[c], ebuf[p], sem_i[p]).wait()
                for k in range(CH // L):
                    sl = pl.ds(k * L, L)
                    t16 = plsc.load_gather(types_v, [ebuf[p][0, sl]])
                    ebuf[p][1, sl] = ebuf[p][1, sl] * T + t16

                @pl.when(j >= 2)
                def _():
                    pltpu.make_async_copy(
                        onesb, hist_sh.at[ebuf[pf].at[1]], sem_s[pf]).wait()

                @pl.when(j + 2 < nc_w)
                def _():
                    pltpu.async_copy(earr_hbm.at[c + 2], ebuf[pf], sem_i[pf])

                pltpu.async_copy(onesb, hist_sh.at[ebuf[p].at[1]],
                                 sem_s[p], add=True)
            return ()

        lax.fori_loop(0, nc_w // 4, body, ())
        for p in ((nc_w - 2) % 4, (nc_w - 1) % 4):
            pltpu.make_async_copy(onesb, hist_sh.at[ebuf[p].at[1]],
                                  sem_s[p]).wait()
        plsc.subcore_barrier()
        pltpu.sync_copy(hist_sh.at[pl.ds(sid * per_tile, per_tile)],
                        out_hbm.at[cid, pl.ds(sid * per_tile, per_tile)])

    return hist


def _make_agg2(N, HH, n_chunks):
    assert n_chunks % NS == 0
    nc_t = n_chunks // NS
    assert nc_t % 4 == 0
    assert N % NS == 0
    rows_per_tile = N // NS
    nfull = rows_per_tile // CH
    rem = rows_per_tile - nfull * CH

    @functools.partial(
        pl.kernel,
        out_type=jax.ShapeDtypeStruct((NC, N, HH), jnp.float32),
        mesh=_mesh(),
        compiler_params=_SC_PARAMS,
        scratch_types=[
            pltpu.VMEM_SHARED((N + 8, HH), jnp.float32),
        ] + [pltpu.VMEM((2, CH), jnp.int32) for _ in range(4)]
          + [pltpu.VMEM((CH, HH), jnp.float32) for _ in range(4)]
          + [pltpu.SemaphoreType.DMA for _ in range(12)],
    )
    def agg2(x1s_hbm, earr_hbm, out_hbm, acc_sh,
             eb0, eb1, eb2, eb3, rw0, rw1, rw2, rw3,
             si0, si1, si2, si3, sg0, sg1, sg2, sg3, ss0, ss1, ss2, ss3):
        ebuf = [eb0, eb1, eb2, eb3]
        rows = [rw0, rw1, rw2, rw3]
        sem_i = [si0, si1, si2, si3]
        sem_g = [sg0, sg1, sg2, sg3]
        sem_s = [ss0, ss1, ss2, ss3]
        cid = lax.axis_index("c")
        sid = lax.axis_index("s")
        row0 = sid * rows_per_tile
        half0 = cid * N
        c0 = sid * nc_t

        for r in range(CH):
            for k in range(HH // L):
                rw0[r, pl.ds(k * L, L)] = jnp.zeros((L,), jnp.float32)
        for q in range(nfull):
            pltpu.sync_copy(rw0, acc_sh.at[pl.ds(row0 + q * CH, CH)])
        if rem:
            pltpu.sync_copy(rw0.at[pl.ds(0, rem)],
                            acc_sh.at[pl.ds(row0 + nfull * CH, rem)])
        plsc.subcore_barrier()

        pltpu.async_copy(earr_hbm.at[c0], ebuf[0], sem_i[0])
        pltpu.async_copy(earr_hbm.at[c0 + 1], ebuf[1], sem_i[1])

        def body(jj, _):
            for p in range(4):
                j = jj * 4 + p
                c = c0 + j
                pm1 = (p + 3) % 4
                pf = (p + 2) % 4
                pltpu.make_async_copy(earr_hbm.at[c], ebuf[p], sem_i[p]).wait()
                for k in range(CH // L):
                    sl = pl.ds(k * L, L)
                    ebuf[p][0, sl] = ebuf[p][0, sl] + half0
                pltpu.async_copy(x1s_hbm.at[ebuf[p].at[0]], rows[p], sem_g[p])

                @pl.when(j >= 1)
                def _():
                    pltpu.make_async_copy(x1s_hbm.at[ebuf[pm1].at[0]],
                                          rows[pm1], sem_g[pm1]).wait()
                    pltpu.async_copy(rows[pm1], acc_sh.at[ebuf[pm1].at[1]],
                                     sem_s[pm1], add=True)

                @pl.when(j >= 2)
                def _():
                    pltpu.make_async_copy(
                        rows[pf], acc_sh.at[ebuf[pf].at[1]], sem_s[pf]).wait()

                @pl.when(j + 2 < nc_t)
                def _():
                    pltpu.async_copy(earr_hbm.at[c + 2], ebuf[pf], sem_i[pf])
            return ()

        lax.fori_loop(0, nc_t // 4, body, ())
        pl_last = (nc_t - 1) % 4
        pltpu.make_async_copy(x1s_hbm.at[ebuf[pl_last].at[0]],
                              rows[pl_last], sem_g[pl_last]).wait()
        pltpu.async_copy(rows[pl_last], acc_sh.at[ebuf[pl_last].at[1]],
                         sem_s[pl_last], add=True)
        for p in ((nc_t - 2) % 4, (nc_t - 1) % 4):
            pltpu.make_async_copy(rows[p], acc_sh.at[ebuf[p].at[1]],
                                  sem_s[p]).wait()
        plsc.subcore_barrier()
        pltpu.sync_copy(acc_sh.at[pl.ds(row0, rows_per_tile)],
                        out_hbm.at[cid, pl.ds(row0, rows_per_tile)])

    return agg2


def _make_qgather(EQ_pad, H, n_chunks, CQ):
    assert n_chunks % NW == 0
    nc_w = n_chunks // NW
    assert nc_w % 4 == 0

    @functools.partial(
        pl.kernel,
        out_type=jax.ShapeDtypeStruct((EQ_pad, H), jnp.float32),
        mesh=_mesh(),
        compiler_params=_SC_PARAMS,
        scratch_types=[pltpu.VMEM((2, CQ), jnp.int32) for _ in range(4)]
                      + [pltpu.VMEM((CQ, H), jnp.float32) for _ in range(8)]
                      + [pltpu.SemaphoreType.DMA for _ in range(12)],
    )
    def qgather(qa_hbm, qb_hbm, qarr_hbm, s_hbm,
                qb0, qb1, qb2, qb3, ua0, ua1, ua2, ua3, ub0, ub1, ub2, ub3,
                si0, si1, si2, si3, sg0, sg1, sg2, sg3, sw0, sw1, sw2, sw3):
        qbuf = [qb0, qb1, qb2, qb3]
        ua = [ua0, ua1, ua2, ua3]
        ub = [ub0, ub1, ub2, ub3]
        sem_i = [si0, si1, si2, si3]
        sem_g = [sg0, sg1, sg2, sg3]
        sem_w = [sw0, sw1, sw2, sw3]

        def finish(pm1, cprev):
            pltpu.make_async_copy(qa_hbm.at[qbuf[pm1].at[0]], ua[pm1],
                                  sem_g[pm1]).wait()
            pltpu.make_async_copy(qb_hbm.at[qbuf[pm1].at[1]], ub[pm1],
                                  sem_g[pm1]).wait()
            for r in range(CQ):
                for k in range(H // L):
                    sl = pl.ds(k * L, L)
                    ua[pm1][r, sl] = ua[pm1][r, sl] + ub[pm1][r, sl]
            pltpu.async_copy(ua[pm1], s_hbm.at[pl.ds(cprev * CQ, CQ)],
                             sem_w[pm1])
        cid = lax.axis_index("c")
        sid = lax.axis_index("s")
        wid = sid * NC + cid
        c0 = wid * nc_w

        pltpu.async_copy(qarr_hbm.at[c0], qbuf[0], sem_i[0])
        pltpu.async_copy(qarr_hbm.at[c0 + 1], qbuf[1], sem_i[1])

        def body(jj, _):
            for p in range(4):
                j = jj * 4 + p
                c = c0 + j
                pm1 = (p + 3) % 4
                pf = (p + 2) % 4
                pltpu.make_async_copy(qarr_hbm.at[c], qbuf[p], sem_i[p]).wait()
                pltpu.async_copy(qa_hbm.at[qbuf[p].at[0]], ua[p], sem_g[p])
                pltpu.async_copy(qb_hbm.at[qbuf[p].at[1]], ub[p], sem_g[p])

                @pl.when(j >= 1)
                def _():
                    finish(pm1, c - 1)

                @pl.when(j >= 2)
                def _():
                    pltpu.make_async_copy(
                        ua[pf], s_hbm.at[pl.ds((c - 2) * CQ, CQ)],
                        sem_w[pf]).wait()

                @pl.when(j + 2 < nc_w)
                def _():
                    pltpu.async_copy(qarr_hbm.at[c + 2], qbuf[pf], sem_i[pf])
            return ()

        lax.fori_loop(0, nc_w // 4, body, ())
        finish((nc_w - 1) % 4, c0 + nc_w - 1)
        for p in ((nc_w - 2) % 4, (nc_w - 1) % 4):
            c = c0 + nc_w - 1
            pltpu.make_async_copy(ua[p], s_hbm.at[pl.ds(c * CQ, CQ)],
                                  sem_w[p]).wait()

    return qgather


def _mlp1_body(h0_ref, h1_ref, nt_ref, te_ref, wg1_ref, bg1_ref,
               x1_ref, den_ref):
    counts = h0_ref[...] + h1_ref[...]
    deg = jnp.sum(counts, axis=1, keepdims=True)
    denom = jnp.maximum(deg, 1.0)
    onehot = (nt_ref[...] == lax.broadcasted_iota(jnp.int32, (1, 8), 1)
              ).astype(jnp.float32)
    cprime = onehot + counts / denom
    te = te_ref[...]
    xn = cprime[:, 0:1] * te[0:1, :]
    for t in range(1, 8):
        xn = xn + cprime[:, t:t + 1] * te[t:t + 1, :]
    x1 = jnp.maximum(
        jnp.dot(xn, wg1_ref[...], preferred_element_type=jnp.float32)
        + bg1_ref[...], 0.0)
    hh = x1.shape[1] // 2
    x1_ref[...] = jnp.stack([x1[:, :hh], x1[:, hh:]], axis=0)
    den_ref[...] = denom


def _mlp2_body(x1_ref, agg_ref, den_ref, wg2_ref, bg2_ref,
               wa_ref, wb_ref, be1_ref, qa_ref, qb_ref):
    x1blk = x1_ref[...]
    aggblk = agg_ref[...]
    x1 = jnp.concatenate([x1blk[0], x1blk[1]], axis=1)
    agg = jnp.concatenate([aggblk[0], aggblk[1]], axis=1)
    xn = x1 + agg / den_ref[...]
    x2 = jnp.maximum(
        jnp.dot(xn, wg2_ref[...], preferred_element_type=jnp.float32)
        + bg2_ref[...], 0.0)
    qa_ref[...] = jnp.dot(x2, wa_ref[...], preferred_element_type=jnp.float32)
    qb_ref[...] = (jnp.dot(x2, wb_ref[...], preferred_element_type=jnp.float32)
                   + be1_ref[...])


def _score_body(s_ref, w2_ref, be2_ref, o_ref):
    h = jnp.maximum(s_ref[...], 0.0)
    o_ref[...] = (jnp.dot(h, w2_ref[...], preferred_element_type=jnp.float32)
                  + be2_ref[...])


def _full(shape):
    return pl.BlockSpec(shape, lambda i: (0,) * len(shape))


def kernel(node_type_idx, edge_index, query_edges, type_embed,
           Wg1, bg1, Wg2, bg2, We1, be1, We2, be2):
    N = node_type_idx.shape[0]
    E = edge_index.shape[1]
    EQ = query_edges.shape[1]
    H = type_embed.shape[1]
    T = type_embed.shape[0]
    HH = H // 2
    R = 2000
    assert N % R == 0

    n_chunks = -(-E // (CH * NS * 4)) * (NS * 4)
    assert n_chunks % NW == 0 and (n_chunks // NW) % 4 == 0
    E_pad = n_chunks * CH
    src_p = jnp.concatenate(
        [edge_index[0], jnp.zeros((E_pad - E,), jnp.int32)])
    dst_p = jnp.concatenate(
        [edge_index[1], jnp.full((E_pad - E,), N, jnp.int32)])
    earr = jnp.stack([src_p.reshape(n_chunks, CH),
                      dst_p.reshape(n_chunks, CH)], axis=1)

    CQ = 80
    nq_chunks = -(-EQ // (CQ * NW * 4)) * (NW * 4)
    EQ_pad = nq_chunks * CQ
    q0_p = jnp.concatenate(
        [query_edges[0], jnp.zeros((EQ_pad - EQ,), jnp.int32)])
    q1_p = jnp.concatenate(
        [query_edges[1], jnp.zeros((EQ_pad - EQ,), jnp.int32)])
    qarr = jnp.stack([q0_p.reshape(nq_chunks, CQ),
                      q1_p.reshape(nq_chunks, CQ)], axis=1)

    hist = _make_hist(N, T, n_chunks)(
        node_type_idx, earr, jnp.zeros((N * T,), jnp.float32))
    h0 = hist[0].reshape(N, T)
    h1 = hist[1].reshape(N, T)

    nt2 = node_type_idx.reshape(N, 1)
    x1, denom = pl.pallas_call(
        _mlp1_body,
        grid=(N // R,),
        in_specs=[
            pl.BlockSpec((R, T), lambda i: (i, 0)),
            pl.BlockSpec((R, T), lambda i: (i, 0)),
            pl.BlockSpec((R, 1), lambda i: (i, 0)),
            _full((T, H)),
            _full((H, H)),
            _full((1, H)),
        ],
        out_specs=[
            pl.BlockSpec((2, R, HH), lambda i: (0, i, 0)),
            pl.BlockSpec((R, 1), lambda i: (i, 0)),
        ],
        out_shape=[
            jax.ShapeDtypeStruct((2, N, HH), jnp.float32),
            jax.ShapeDtypeStruct((N, 1), jnp.float32),
        ],
    )(h0, h1, nt2, type_embed, Wg1, bg1.reshape(1, H))

    agg = _make_agg2(N, HH, n_chunks)(x1.reshape(2 * N, HH), earr)

    qa, qb = pl.pallas_call(
        _mlp2_body,
        grid=(N // R,),
        in_specs=[
            pl.BlockSpec((2, R, HH), lambda i: (0, i, 0)),
            pl.BlockSpec((2, R, HH), lambda i: (0, i, 0)),
            pl.BlockSpec((R, 1), lambda i: (i, 0)),
            _full((H, H)),
            _full((1, H)),
            _full((H, H)),
            _full((H, H)),
            _full((1, H)),
        ],
        out_specs=[
            pl.BlockSpec((R, H), lambda i: (i, 0)),
            pl.BlockSpec((R, H), lambda i: (i, 0)),
        ],
        out_shape=[
            jax.ShapeDtypeStruct((N, H), jnp.float32),
            jax.ShapeDtypeStruct((N, H), jnp.float32),
        ],
    )(x1, agg, denom, Wg2, bg2.reshape(1, H),
      We1[:H], We1[H:], be1.reshape(1, H))

    s = _make_qgather(EQ_pad, H, nq_chunks, CQ)(qa, qb, qarr)

    RQ = 2048
    assert EQ_pad % RQ == 0
    logits = pl.pallas_call(
        _score_body,
        grid=(EQ_pad // RQ,),
        in_specs=[
            pl.BlockSpec((RQ, H), lambda i: (i, 0)),
            _full((H, 1)),
            _full((1, 1)),
        ],
        out_specs=pl.BlockSpec((RQ, 1), lambda i: (i, 0)),
        out_shape=jax.ShapeDtypeStruct((EQ_pad, 1), jnp.float32),
    )(s, We2, be2.reshape(1, 1))

    return logits.reshape(EQ_pad)[:EQ]


# speedup vs baseline: 10.0853x; 1.0567x over previous
import functools

import jax
import jax.numpy as jnp
from jax import lax
from jax.experimental import pallas as pl
from jax.experimental.pallas import tpu as pltpu
from jax.experimental.pallas import tpu_sc as plsc

NC = 2
NS = 16
NW = NC * NS
L = 16
CH = 128

_SC_PARAMS = pltpu.CompilerParams(use_tc_tiling_on_sc=False,
                                  needs_layout_passes=False)


def _mesh():
    return plsc.VectorSubcoreMesh(core_axis_name="c", subcore_axis_name="s",
                                  num_cores=NC, num_subcores=NS)


def _make_hist(N, T, n_chunks):
    assert n_chunks % NW == 0
    nc_w = n_chunks // NW
    assert nc_w % 4 == 0
    NK = N * T
    assert NK % NS == 0
    per_tile = NK // NS

    @functools.partial(
        pl.kernel,
        out_type=jax.ShapeDtypeStruct((NC, NK), jnp.float32),
        mesh=_mesh(),
        compiler_params=_SC_PARAMS,
        scratch_types=[
            pltpu.VMEM((N,), jnp.int32),
            pltpu.VMEM((CH,), jnp.float32),
            pltpu.VMEM_SHARED((NK + T,), jnp.float32),
        ] + [pltpu.VMEM((2, CH), jnp.int32) for _ in range(4)]
          + [pltpu.SemaphoreType.DMA for _ in range(8)],
    )
    def hist(types_hbm, earr_hbm, zeros_hbm, out_hbm,
             types_v, onesb, hist_sh,
             eb0, eb1, eb2, eb3, si0, si1, si2, si3, ss0, ss1, ss2, ss3):
        ebuf = [eb0, eb1, eb2, eb3]
        sem_i = [si0, si1, si2, si3]
        sem_s = [ss0, ss1, ss2, ss3]
        cid = lax.axis_index("c")
        sid = lax.axis_index("s")
        wid = sid * NC + cid
        c0 = wid * nc_w

        pltpu.sync_copy(types_hbm, types_v)
        for k in range(CH // L):
            onesb[pl.ds(k * L, L)] = jnp.full((L,), 1.0, jnp.float32)
        pltpu.sync_copy(zeros_hbm.at[pl.ds(sid * per_tile, per_tile)],
                        hist_sh.at[pl.ds(sid * per_tile, per_tile)])
        plsc.subcore_barrier()

        pltpu.async_copy(earr_hbm.at[c0], ebuf[0], sem_i[0])
        pltpu.async_copy(earr_hbm.at[c0 + 1], ebuf[1], sem_i[1])

        def body(jj, _):
            for p in range(4):
                j = jj * 4 + p
                c = c0 + j
                pf = (p + 2) % 4
                pltpu.make_async_copy(earr_hbm.at[c], ebuf[p], sem_i[p]).wait()
                for k in range(CH // L):
                    sl = pl.ds(k * L, L)
                    t16 = plsc.load_gather(types_v, [ebuf[p][0, sl]])
                    ebuf[p][1, sl] = ebuf[p][1, sl] * T + t16

                @pl.when(j >= 2)
                def _():
                    pltpu.make_async_copy(
                        onesb, hist_sh.at[ebuf[pf].at[1]], sem_s[pf]).wait()

                @pl.when(j + 2 < nc_w)
                def _():
                    pltpu.async_copy(earr_hbm.at[c + 2], ebuf[pf], sem_i[pf])

                pltpu.async_copy(onesb, hist_sh.at[ebuf[p].at[1]],
                                 sem_s[p], add=True)
            return ()

        lax.fori_loop(0, nc_w // 4, body, ())
        for p in ((nc_w - 2) % 4, (nc_w - 1) % 4):
            pltpu.make_async_copy(onesb, hist_sh.at[ebuf[p].at[1]],
                                  sem_s[p]).wait()
        plsc.subcore_barrier()
        pltpu.sync_copy(hist_sh.at[pl.ds(sid * per_tile, per_tile)],
                        out_hbm.at[cid, pl.ds(sid * per_tile, per_tile)])

    return hist


def _make_agg2(N, HH, n_chunks):
    assert n_chunks % NS == 0
    nc_t = n_chunks // NS
    NSLOT = 7
    assert nc_t % NSLOT == 0
    assert N % NS == 0
    rows_per_tile = N // NS
    nfull = rows_per_tile // CH
    rem = rows_per_tile - nfull * CH

    @functools.partial(
        pl.kernel,
        out_type=jax.ShapeDtypeStruct((NC, N, HH), jnp.float32),
        mesh=_mesh(),
        compiler_params=_SC_PARAMS,
        scratch_types=[
            pltpu.VMEM_SHARED((N + 8, HH), jnp.float32),
        ] + [pltpu.VMEM((2, CH), jnp.int32) for _ in range(NSLOT)]
          + [pltpu.VMEM((CH, HH), jnp.float32) for _ in range(NSLOT)]
          + [pltpu.SemaphoreType.DMA for _ in range(3 * NSLOT)],
    )
    def agg2(x1s_hbm, earr_hbm, out_hbm, acc_sh, *bufs):
        ebuf = list(bufs[0:NSLOT])
        rows = list(bufs[NSLOT:2 * NSLOT])
        sem_i = list(bufs[2 * NSLOT:3 * NSLOT])
        sem_g = list(bufs[3 * NSLOT:4 * NSLOT])
        sem_s = list(bufs[4 * NSLOT:5 * NSLOT])
        cid = lax.axis_index("c")
        sid = lax.axis_index("s")
        row0 = sid * rows_per_tile
        half0 = cid * N
        c0 = sid * nc_t

        for r in range(CH):
            for k in range(HH // L):
                rows[0][r, pl.ds(k * L, L)] = jnp.zeros((L,), jnp.float32)
        for q in range(nfull):
            pltpu.sync_copy(rows[0], acc_sh.at[pl.ds(row0 + q * CH, CH)])
        if rem:
            pltpu.sync_copy(rows[0].at[pl.ds(0, rem)],
                            acc_sh.at[pl.ds(row0 + nfull * CH, rem)])
        plsc.subcore_barrier()

        for p in range(3):
            pltpu.async_copy(earr_hbm.at[c0 + p], ebuf[p], sem_i[p])

        def body(jj, _):
            for p in range(NSLOT):
                j = jj * NSLOT + p
                c = c0 + j
                pm2 = (p + NSLOT - 2) % NSLOT
                pm4 = (p + NSLOT - 4) % NSLOT
                pf = (p + 3) % NSLOT
                pltpu.make_async_copy(earr_hbm.at[c], ebuf[p], sem_i[p]).wait()
                for k in range(CH // L):
                    sl = pl.ds(k * L, L)
                    ebuf[p][0, sl] = ebuf[p][0, sl] + half0
                pltpu.async_copy(x1s_hbm.at[ebuf[p].at[0]], rows[p], sem_g[p])

                @pl.when(j >= 2)
                def _():
                    pltpu.make_async_copy(x1s_hbm.at[ebuf[pm2].at[0]],
                                          rows[pm2], sem_g[pm2]).wait()
                    pltpu.async_copy(rows[pm2], acc_sh.at[ebuf[pm2].at[1]],
                                     sem_s[pm2], add=True)

                @pl.when(j >= 4)
                def _():
                    pltpu.make_async_copy(
                        rows[pm4], acc_sh.at[ebuf[pm4].at[1]], sem_s[pm4]).wait()

                @pl.when(j + 3 < nc_t)
                def _():
                    pltpu.async_copy(earr_hbm.at[c + 3], ebuf[pf], sem_i[pf])
            return ()

        lax.fori_loop(0, nc_t // NSLOT, body, ())
        for d in (2, 1):
            p = (nc_t - d) % NSLOT
            pltpu.make_async_copy(x1s_hbm.at[ebuf[p].at[0]],
                                  rows[p], sem_g[p]).wait()
            pltpu.async_copy(rows[p], acc_sh.at[ebuf[p].at[1]],
                             sem_s[p], add=True)
        for d in (4, 3, 2, 1):
            p = (nc_t - d) % NSLOT
            pltpu.make_async_copy(rows[p], acc_sh.at[ebuf[p].at[1]],
                                  sem_s[p]).wait()
        plsc.subcore_barrier()
        pltpu.sync_copy(acc_sh.at[pl.ds(row0, rows_per_tile)],
                        out_hbm.at[cid, pl.ds(row0, rows_per_tile)])

    return agg2


def _make_qgather(EQ_pad, H, n_chunks, CQ):
    assert n_chunks % NW == 0
    nc_w = n_chunks // NW
    assert nc_w % 4 == 0

    @functools.partial(
        pl.kernel,
        out_type=jax.ShapeDtypeStruct((EQ_pad, H), jnp.float32),
        mesh=_mesh(),
        compiler_params=_SC_PARAMS,
        scratch_types=[pltpu.VMEM((2, CQ), jnp.int32) for _ in range(4)]
                      + [pltpu.VMEM((CQ, H), jnp.float32) for _ in range(8)]
                      + [pltpu.SemaphoreType.DMA for _ in range(12)],
    )
    def qgather(qa_hbm, qb_hbm, qarr_hbm, s_hbm,
                qb0, qb1, qb2, qb3, ua0, ua1, ua2, ua3, ub0, ub1, ub2, ub3,
                si0, si1, si2, si3, sg0, sg1, sg2, sg3, sw0, sw1, sw2, sw3):
        qbuf = [qb0, qb1, qb2, qb3]
        ua = [ua0, ua1, ua2, ua3]
        ub = [ub0, ub1, ub2, ub3]
        sem_i = [si0, si1, si2, si3]
        sem_g = [sg0, sg1, sg2, sg3]
        sem_w = [sw0, sw1, sw2, sw3]

        def finish(pm1, cprev):
            pltpu.make_async_copy(qa_hbm.at[qbuf[pm1].at[0]], ua[pm1],
                                  sem_g[pm1]).wait()
            pltpu.make_async_copy(qb_hbm.at[qbuf[pm1].at[1]], ub[pm1],
                                  sem_g[pm1]).wait()
            for r in range(CQ):
                for k in range(H // L):
                    sl = pl.ds(k * L, L)
                    ua[pm1][r, sl] = ua[pm1][r, sl] + ub[pm1][r, sl]
            pltpu.async_copy(ua[pm1], s_hbm.at[pl.ds(cprev * CQ, CQ)],
                             sem_w[pm1])
        cid = lax.axis_index("c")
        sid = lax.axis_index("s")
        wid = sid * NC + cid
        c0 = wid * nc_w

        pltpu.async_copy(qarr_hbm.at[c0], qbuf[0], sem_i[0])
        pltpu.async_copy(qarr_hbm.at[c0 + 1], qbuf[1], sem_i[1])

        def body(jj, _):
            for p in range(4):
                j = jj * 4 + p
                c = c0 + j
                pm1 = (p + 3) % 4
                pf = (p + 2) % 4
                pltpu.make_async_copy(qarr_hbm.at[c], qbuf[p], sem_i[p]).wait()
                pltpu.async_copy(qa_hbm.at[qbuf[p].at[0]], ua[p], sem_g[p])
                pltpu.async_copy(qb_hbm.at[qbuf[p].at[1]], ub[p], sem_g[p])

                @pl.when(j >= 1)
                def _():
                    finish(pm1, c - 1)

                @pl.when(j >= 2)
                def _():
                    pltpu.make_async_copy(
                        ua[pf], s_hbm.at[pl.ds((c - 2) * CQ, CQ)],
                        sem_w[pf]).wait()

                @pl.when(j + 2 < nc_w)
                def _():
                    pltpu.async_copy(qarr_hbm.at[c + 2], qbuf[pf], sem_i[pf])
            return ()

        lax.fori_loop(0, nc_w // 4, body, ())
        finish((nc_w - 1) % 4, c0 + nc_w - 1)
        for p in ((nc_w - 2) % 4, (nc_w - 1) % 4):
            c = c0 + nc_w - 1
            pltpu.make_async_copy(ua[p], s_hbm.at[pl.ds(c * CQ, CQ)],
                                  sem_w[p]).wait()

    return qgather


def _mlp1_body(h0_ref, h1_ref, nt_ref, te_ref, wg1_ref, bg1_ref,
               x1_ref, den_ref):
    counts = h0_ref[...] + h1_ref[...]
    deg = jnp.sum(counts, axis=1, keepdims=True)
    denom = jnp.maximum(deg, 1.0)
    onehot = (nt_ref[...] == lax.broadcasted_iota(jnp.int32, (1, 8), 1)
              ).astype(jnp.float32)
    cprime = onehot + counts / denom
    te = te_ref[...]
    xn = cprime[:, 0:1] * te[0:1, :]
    for t in range(1, 8):
        xn = xn + cprime[:, t:t + 1] * te[t:t + 1, :]
    x1 = jnp.maximum(
        jnp.dot(xn, wg1_ref[...], preferred_element_type=jnp.float32)
        + bg1_ref[...], 0.0)
    hh = x1.shape[1] // 2
    x1_ref[...] = jnp.stack([x1[:, :hh], x1[:, hh:]], axis=0)
    den_ref[...] = denom


def _mlp2_body(x1_ref, agg_ref, den_ref, wg2_ref, bg2_ref,
               wa_ref, wb_ref, be1_ref, qa_ref, qb_ref):
    x1blk = x1_ref[...]
    aggblk = agg_ref[...]
    x1 = jnp.concatenate([x1blk[0], x1blk[1]], axis=1)
    agg = jnp.concatenate([aggblk[0], aggblk[1]], axis=1)
    xn = x1 + agg / den_ref[...]
    x2 = jnp.maximum(
        jnp.dot(xn, wg2_ref[...], preferred_element_type=jnp.float32)
        + bg2_ref[...], 0.0)
    qa_ref[...] = jnp.dot(x2, wa_ref[...], preferred_element_type=jnp.float32)
    qb_ref[...] = (jnp.dot(x2, wb_ref[...], preferred_element_type=jnp.float32)
                   + be1_ref[...])


def _score_body(s_ref, w2_ref, be2_ref, o_ref):
    h = jnp.maximum(s_ref[...], 0.0)
    o_ref[...] = (jnp.dot(h, w2_ref[...], preferred_element_type=jnp.float32)
                  + be2_ref[...])


def _full(shape):
    return pl.BlockSpec(shape, lambda i: (0,) * len(shape))


def kernel(node_type_idx, edge_index, query_edges, type_embed,
           Wg1, bg1, Wg2, bg2, We1, be1, We2, be2):
    N = node_type_idx.shape[0]
    E = edge_index.shape[1]
    EQ = query_edges.shape[1]
    H = type_embed.shape[1]
    T = type_embed.shape[0]
    HH = H // 2
    R = 2000
    assert N % R == 0

    n_chunks = -(-E // (CH * NS * 4)) * (NS * 4)
    assert n_chunks % NW == 0 and (n_chunks // NW) % 4 == 0
    E_pad = n_chunks * CH
    src_p = jnp.concatenate(
        [edge_index[0], jnp.zeros((E_pad - E,), jnp.int32)])
    dst_p = jnp.concatenate(
        [edge_index[1], jnp.full((E_pad - E,), N, jnp.int32)])
    earr = jnp.stack([src_p.reshape(n_chunks, CH),
                      dst_p.reshape(n_chunks, CH)], axis=1)

    CQ = 80
    nq_chunks = -(-EQ // (CQ * NW * 4)) * (NW * 4)
    EQ_pad = nq_chunks * CQ
    q0_p = jnp.concatenate(
        [query_edges[0], jnp.zeros((EQ_pad - EQ,), jnp.int32)])
    q1_p = jnp.concatenate(
        [query_edges[1], jnp.zeros((EQ_pad - EQ,), jnp.int32)])
    qarr = jnp.stack([q0_p.reshape(nq_chunks, CQ),
                      q1_p.reshape(nq_chunks, CQ)], axis=1)

    hist = _make_hist(N, T, n_chunks)(
        node_type_idx, earr, jnp.zeros((N * T,), jnp.float32))
    h0 = hist[0].reshape(N, T)
    h1 = hist[1].reshape(N, T)

    nt2 = node_type_idx.reshape(N, 1)
    x1, denom = pl.pallas_call(
        _mlp1_body,
        grid=(N // R,),
        in_specs=[
            pl.BlockSpec((R, T), lambda i: (i, 0)),
            pl.BlockSpec((R, T), lambda i: (i, 0)),
            pl.BlockSpec((R, 1), lambda i: (i, 0)),
            _full((T, H)),
            _full((H, H)),
            _full((1, H)),
        ],
        out_specs=[
            pl.BlockSpec((2, R, HH), lambda i: (0, i, 0)),
            pl.BlockSpec((R, 1), lambda i: (i, 0)),
        ],
        out_shape=[
            jax.ShapeDtypeStruct((2, N, HH), jnp.float32),
            jax.ShapeDtypeStruct((N, 1), jnp.float32),
        ],
    )(h0, h1, nt2, type_embed, Wg1, bg1.reshape(1, H))

    agg = _make_agg2(N, HH, n_chunks)(x1.reshape(2 * N, HH), earr)

    qa, qb = pl.pallas_call(
        _mlp2_body,
        grid=(N // R,),
        in_specs=[
            pl.BlockSpec((2, R, HH), lambda i: (0, i, 0)),
            pl.BlockSpec((2, R, HH), lambda i: (0, i, 0)),
            pl.BlockSpec((R, 1), lambda i: (i, 0)),
            _full((H, H)),
            _full((1, H)),
            _full((H, H)),
            _full((H, H)),
            _full((1, H)),
        ],
        out_specs=[
            pl.BlockSpec((R, H), lambda i: (i, 0)),
            pl.BlockSpec((R, H), lambda i: (i, 0)),
        ],
        out_shape=[
            jax.ShapeDtypeStruct((N, H), jnp.float32),
            jax.ShapeDtypeStruct((N, H), jnp.float32),
        ],
    )(x1, agg, denom, Wg2, bg2.reshape(1, H),
      We1[:H], We1[H:], be1.reshape(1, H))

    s = _make_qgather(EQ_pad, H, nq_chunks, CQ)(qa, qb, qarr)

    RQ = 2048
    assert EQ_pad % RQ == 0
    logits = pl.pallas_call(
        _score_body,
        grid=(EQ_pad // RQ,),
        in_specs=[
            pl.BlockSpec((RQ, H), lambda i: (i, 0)),
            _full((H, 1)),
            _full((1, 1)),
        ],
        out_specs=pl.BlockSpec((RQ, 1), lambda i: (i, 0)),
        out_shape=jax.ShapeDtypeStruct((EQ_pad, 1), jnp.float32),
    )(s, We2, be2.reshape(1, 1))

    return logits.reshape(EQ_pad)[:EQ]
